# baseline (speedup 1.0000x reference)
import functools

import jax
import jax.numpy as jnp
from jax import lax
from jax.experimental import pallas as pl
from jax.experimental.pallas import tpu as pltpu
from jax.experimental.pallas import tpu_sc as plsc

N_NODES = 10000
D = 128
DA = 16
E = 320000

NC = 2
NS = 16
NW = NC * NS
E_H = E // 2
E_PER = E_H // NW
CH = 40
NCH = E_PER // CH
NBUF = 4
APAD = 10240
ROWS_PER = APAD // NS

_INV_SQRT_D = 1.0 / (D ** 0.5)
_INV_SQRT8 = 1.0 / (8.0 ** 0.5)
_INV_SQRT64 = 1.0 / 8.0


def _silu(x):
    return (0.5 * x) * (1.0 + jnp.tanh(0.5 * x))


def _h_body(nf, w, o):
    o[...] = jnp.dot(nf[...], w[...], preferred_element_type=jnp.float32) * _INV_SQRT_D


def _mlp_body(xT, eaT, w0, w1, w2, w3, o):
    a = lax.dot_general(xT[...], w0[...], (((0,), (0,)), ((), ())),
                        preferred_element_type=jnp.float32)
    a = _silu(a * _INV_SQRT8)
    a = _silu(jnp.dot(a, w1[...], preferred_element_type=jnp.float32) * _INV_SQRT64)
    a = _silu(jnp.dot(a, w2[...], preferred_element_type=jnp.float32) * _INV_SQRT64)
    a = a * jnp.transpose(eaT[...])
    o[...] = jnp.dot(a, w3[...], preferred_element_type=jnp.float32) * _INV_SQRT64


def _final_body(p0, p1, p2, p3, at, wl, ws, o):
    msg = jnp.dot(p0[0] + p1[0] + p2[0] + p3[0], wl[...],
                  preferred_element_type=jnp.float32)
    msg = msg * (_INV_SQRT_D / 32.0)
    attrs = at[...]
    acc = jnp.zeros(msg.shape, jnp.float32)
    for v in range(DA):
        acc = acc + jnp.dot(msg * attrs[:, v:v + 1], ws[v],
                            preferred_element_type=jnp.float32)
    o[...] = acc * (1.0 / ((D * DA) ** 0.5))


def _sc_scatter_body(h_hbm, tp2_hbm, sr_hbm, zero_hbm, out_hbm,
                     srb, rows, tp2v, acc, isems, gsems, tsems, ssems):
    cid = lax.axis_index("c")
    sid = lax.axis_index("s")
    wid = sid * NC + cid

    pltpu.sync_copy(zero_hbm, acc.at[pl.ds(sid * ROWS_PER, ROWS_PER)])
    plsc.subcore_barrier()

    cbase = wid * NCH

    def mul(b):
        @plsc.parallel_loop(0, CH, unroll=4)
        def _(r):
            for j in range(D // 16):
                s = pl.ds(j * 16, 16)
                rows[b, r, s] = rows[b, r, s] * tp2v[b, r, s]

    def stage_a(b, c, first=False):
        if not first:
            pltpu.make_async_copy(rows.at[b], acc.at[srb.at[b, 0]],
                                  ssems.at[b]).wait()
        pltpu.async_copy(sr_hbm.at[cbase + c], srb.at[b], isems.at[b])
        pltpu.async_copy(tp2_hbm.at[pl.ds((cbase + c) * CH, CH)], tp2v.at[b],
                         tsems.at[b])

    def stage_b(b, c):
        pltpu.make_async_copy(sr_hbm.at[cbase], srb.at[b], isems.at[b]).wait()
        pltpu.async_copy(h_hbm.at[srb.at[b, 1]], rows.at[b], gsems.at[b])

    def stage_c(b):
        pltpu.make_async_copy(h_hbm.at[srb.at[b, 1]], rows.at[b],
                              gsems.at[b]).wait()
        pltpu.make_async_copy(tp2_hbm.at[pl.ds(0, CH)], tp2v.at[b],
                              tsems.at[b]).wait()
        mul(b)
        pltpu.async_copy(rows.at[b], acc.at[srb.at[b, 0]], ssems.at[b],
                         add=True)

    stage_a(0, 0, first=True)
    stage_a(1, 1, first=True)
    stage_b(0, 0)
    stage_a(2, 2, first=True)
    stage_b(1, 1)
    stage_c(0)
    stage_a(3, 3, first=True)
    stage_b(2, 2)
    stage_c(1)

    def quad(i, _):
        t = 4 * i + 2
        stage_a(0, t + 2)
        stage_b(3, t + 1)
        stage_c(2)
        stage_a(1, t + 3)
        stage_b(0, t + 2)
        stage_c(3)
        stage_a(2, t + 4)
        stage_b(1, t + 3)
        stage_c(0)
        stage_a(3, t + 5)
        stage_b(2, t + 4)
        stage_c(1)
        return 0

    lax.fori_loop(0, 30, quad, 0)

    stage_a(0, 124)
    stage_b(3, 123)
    stage_c(2)
    stage_b(0, 124)
    stage_c(3)
    stage_c(0)
    for b in (1, 2, 3, 0):
        pltpu.make_async_copy(rows.at[b], acc.at[srb.at[b, 0]],
                              ssems.at[b]).wait()

    plsc.subcore_barrier()
    pltpu.sync_copy(acc.at[pl.ds(sid * ROWS_PER, ROWS_PER)],
                    out_hbm.at[cid, pl.ds(sid * ROWS_PER, ROWS_PER)])


_sc_scatter = functools.partial(
    pl.kernel,
    out_type=jax.ShapeDtypeStruct((NC, APAD, D), jnp.float32),
    mesh=plsc.VectorSubcoreMesh(core_axis_name="c", subcore_axis_name="s",
                                num_cores=NC, num_subcores=NS),
    scratch_types=[
        pltpu.VMEM((NBUF, 2, CH), jnp.int32),
        pltpu.VMEM((NBUF, CH, D), jnp.float32),
        pltpu.VMEM((NBUF, CH, D), jnp.float32),
        pltpu.VMEM_SHARED((APAD, D), jnp.float32),
        pltpu.SemaphoreType.DMA((NBUF,)),
        pltpu.SemaphoreType.DMA((NBUF,)),
        pltpu.SemaphoreType.DMA((NBUF,)),
        pltpu.SemaphoreType.DMA((NBUF,)),
    ],
)(_sc_scatter_body)


def kernel(node_attrs, node_feats, edge_attrs, edge_feats, edge_index,
           W_up, mlp_W0, mlp_W1, mlp_W2, mlp_W3, W_lin, W_skip):
    sender = edge_index[0].astype(jnp.int32)
    receiver = edge_index[1].astype(jnp.int32)

    BN = 2000
    h = pl.pallas_call(
        _h_body,
        grid=(N_NODES // BN,),
        in_specs=[pl.BlockSpec((BN, D), lambda i: (i, 0)),
                  pl.BlockSpec((D, D), lambda i: (0, 0))],
        out_specs=pl.BlockSpec((BN, D), lambda i: (i, 0)),
        out_shape=jax.ShapeDtypeStruct((N_NODES, D), jnp.float32),
    )(node_feats, W_up)

    BE = 6400
    NBLK_H = E_H // BE
    efT, eaT = edge_feats.T, edge_attrs.T

    def _mlp_half(half):
        return pl.pallas_call(
            _mlp_body,
            grid=(NBLK_H,),
            in_specs=[pl.BlockSpec((8, BE), lambda i: (0, i + half * NBLK_H)),
                      pl.BlockSpec((1, BE), lambda i: (0, i + half * NBLK_H)),
                      pl.BlockSpec((8, 64), lambda i: (0, 0)),
                      pl.BlockSpec((64, 64), lambda i: (0, 0)),
                      pl.BlockSpec((64, 64), lambda i: (0, 0)),
                      pl.BlockSpec((64, D), lambda i: (0, 0))],
            out_specs=pl.BlockSpec((BE, D), lambda i: (i, 0)),
            out_shape=jax.ShapeDtypeStruct((E_H, D), jnp.float32),
        )(efT, eaT, mlp_W0, mlp_W1, mlp_W2, mlp_W3)

    zeros_blk = jnp.zeros((ROWS_PER, D), jnp.float32)
    sr_full = jnp.stack([receiver.reshape(-1, CH),
                         sender.reshape(-1, CH)], axis=1)
    nch_h = E_H // CH

    tp2_a = _mlp_half(0)
    partials_a = _sc_scatter(h, tp2_a, sr_full[:nch_h], zeros_blk)
    tp2_b = _mlp_half(1)
    partials_b = _sc_scatter(h, tp2_b, sr_full[nch_h:], zeros_blk)

    W_skip_t = jnp.transpose(W_skip, (1, 0, 2))
    BF = 2000
    out = pl.pallas_call(
        _final_body,
        grid=(N_NODES // BF,),
        in_specs=[pl.BlockSpec((1, BF, D), lambda i: (0, i, 0)),
                  pl.BlockSpec((1, BF, D), lambda i: (1, i, 0)),
                  pl.BlockSpec((1, BF, D), lambda i: (0, i, 0)),
                  pl.BlockSpec((1, BF, D), lambda i: (1, i, 0)),
                  pl.BlockSpec((BF, DA), lambda i: (i, 0)),
                  pl.BlockSpec((D, D), lambda i: (0, 0)),
                  pl.BlockSpec((DA, D, D), lambda i: (0, 0, 0))],
        out_specs=pl.BlockSpec((BF, D), lambda i: (i, 0)),
        out_shape=jax.ShapeDtypeStruct((N_NODES, D), jnp.float32),
    )(partials_a, partials_a, partials_b, partials_b,
      node_attrs, W_lin, W_skip_t)

    return out.reshape(N_NODES, D, 1)


# speedup vs baseline: 1.0845x; 1.0845x over previous
import functools

import jax
import jax.numpy as jnp
from jax import lax
from jax.experimental import pallas as pl
from jax.experimental.pallas import tpu as pltpu
from jax.experimental.pallas import tpu_sc as plsc

N_NODES = 10000
D = 128
DA = 16
E = 320000

NC = 2
NS = 16
NW = NC * NS
E_H = E // 2
E_PER = E_H // NW
CH = 40
NCH = E_PER // CH
NBUF = 4
APAD = 10240
ROWS_PER = APAD // NS

_INV_SQRT_D = 1.0 / (D ** 0.5)
_INV_SQRT8 = 1.0 / (8.0 ** 0.5)
_INV_SQRT64 = 1.0 / 8.0


def _silu(x):
    return (0.5 * x) * (1.0 + jnp.tanh(0.5 * x))


def _h_body(nf, w, o):
    o[...] = jnp.dot(nf[...], w[...], preferred_element_type=jnp.float32) * _INV_SQRT_D


def _mlp_body(xT, eaT, w0, w1, w2, w3, o):
    a = lax.dot_general(xT[...], w0[...], (((0,), (0,)), ((), ())),
                        preferred_element_type=jnp.float32)
    a = _silu(a * _INV_SQRT8)
    a = _silu(jnp.dot(a, w1[...], preferred_element_type=jnp.float32) * _INV_SQRT64)
    a = _silu(jnp.dot(a, w2[...], preferred_element_type=jnp.float32) * _INV_SQRT64)
    a = a * jnp.transpose(eaT[...])
    o[...] = jnp.dot(a, w3[...], preferred_element_type=jnp.float32) * _INV_SQRT64


def _final_body(p0, p1, p2, p3, at, wl, ws, o):
    msg = jnp.dot(p0[0] + p1[0] + p2[0] + p3[0], wl[...],
                  preferred_element_type=jnp.float32)
    msg = msg * (_INV_SQRT_D / 32.0)
    attrs = at[...]
    acc = jnp.zeros(msg.shape, jnp.float32)
    for v in range(DA):
        acc = acc + jnp.dot(msg * attrs[:, v:v + 1], ws[v],
                            preferred_element_type=jnp.float32)
    o[...] = acc * (1.0 / ((D * DA) ** 0.5))


def _sc_scatter_body(half, ei_hbm, h_hbm, tp2_hbm, zero_hbm, out_hbm,
                     srb, rows, tp2v, acc, isems, gsems, tsems, ssems):
    cid = lax.axis_index("c")
    sid = lax.axis_index("s")
    wid = sid * NC + cid

    pltpu.sync_copy(zero_hbm, acc.at[pl.ds(sid * ROWS_PER, ROWS_PER)])
    plsc.subcore_barrier()

    tbase = wid * E_PER
    ebase = half * E_H + tbase

    def mul(b):
        @plsc.parallel_loop(0, CH, unroll=2)
        def _(r):
            for j in range(D // 16):
                s = pl.ds(j * 16, 16)
                rows[b, r, s] = rows[b, r, s] * tp2v[b, r, s]

    def stage_a(b, c, first=False):
        if not first:
            pltpu.make_async_copy(rows.at[b], acc.at[srb.at[b, 0]],
                                  ssems.at[b]).wait()
        pltpu.async_copy(ei_hbm.at[pl.ds(E + ebase + c * CH, CH)],
                         srb.at[b, 0], isems.at[b])
        pltpu.async_copy(ei_hbm.at[pl.ds(ebase + c * CH, CH)],
                         srb.at[b, 1], isems.at[b])
        pltpu.async_copy(tp2_hbm.at[pl.ds(tbase + c * CH, CH)], tp2v.at[b],
                         tsems.at[b])

    def stage_b(b, c):
        pltpu.make_async_copy(ei_hbm.at[pl.ds(0, CH)], srb.at[b, 0],
                              isems.at[b]).wait()
        pltpu.make_async_copy(ei_hbm.at[pl.ds(0, CH)], srb.at[b, 1],
                              isems.at[b]).wait()
        pltpu.async_copy(h_hbm.at[srb.at[b, 1]], rows.at[b], gsems.at[b])

    def stage_c(b):
        pltpu.make_async_copy(h_hbm.at[srb.at[b, 1]], rows.at[b],
                              gsems.at[b]).wait()
        pltpu.make_async_copy(tp2_hbm.at[pl.ds(0, CH)], tp2v.at[b],
                              tsems.at[b]).wait()
        mul(b)
        pltpu.async_copy(rows.at[b], acc.at[srb.at[b, 0]], ssems.at[b],
                         add=True)

    stage_a(0, 0, first=True)
    stage_a(1, 1, first=True)
    stage_b(0, 0)
    stage_a(2, 2, first=True)
    stage_b(1, 1)
    stage_c(0)
    stage_a(3, 3, first=True)
    stage_b(2, 2)
    stage_c(1)

    def quad(i, _):
        t = 4 * i + 2
        stage_a(0, t + 2)
        stage_b(3, t + 1)
        stage_c(2)
        stage_a(1, t + 3)
        stage_b(0, t + 2)
        stage_c(3)
        stage_a(2, t + 4)
        stage_b(1, t + 3)
        stage_c(0)
        stage_a(3, t + 5)
        stage_b(2, t + 4)
        stage_c(1)
        return 0

    lax.fori_loop(0, 30, quad, 0)

    stage_a(0, 124)
    stage_b(3, 123)
    stage_c(2)
    stage_b(0, 124)
    stage_c(3)
    stage_c(0)
    for b in (1, 2, 3, 0):
        pltpu.make_async_copy(rows.at[b], acc.at[srb.at[b, 0]],
                              ssems.at[b]).wait()

    plsc.subcore_barrier()
    pltpu.sync_copy(acc.at[pl.ds(sid * ROWS_PER, ROWS_PER)],
                    out_hbm.at[cid, pl.ds(sid * ROWS_PER, ROWS_PER)])


def _make_sc_scatter(half):
    return functools.partial(
        pl.kernel,
        out_type=jax.ShapeDtypeStruct((NC, APAD, D), jnp.float32),
        mesh=plsc.VectorSubcoreMesh(core_axis_name="c", subcore_axis_name="s",
                                    num_cores=NC, num_subcores=NS),
        scratch_types=[
            pltpu.VMEM((NBUF, 2, CH), jnp.int32),
            pltpu.VMEM((NBUF, CH, D), jnp.float32),
            pltpu.VMEM((NBUF, CH, D), jnp.float32),
            pltpu.VMEM_SHARED((APAD, D), jnp.float32),
            pltpu.SemaphoreType.DMA((NBUF,)),
            pltpu.SemaphoreType.DMA((NBUF,)),
            pltpu.SemaphoreType.DMA((NBUF,)),
            pltpu.SemaphoreType.DMA((NBUF,)),
        ],
    )(functools.partial(_sc_scatter_body, half))


_sc_scatter_a = _make_sc_scatter(0)
_sc_scatter_b = _make_sc_scatter(1)


def kernel(node_attrs, node_feats, edge_attrs, edge_feats, edge_index,
           W_up, mlp_W0, mlp_W1, mlp_W2, mlp_W3, W_lin, W_skip):
    BN = 2000
    h = pl.pallas_call(
        _h_body,
        grid=(N_NODES // BN,),
        in_specs=[pl.BlockSpec((BN, D), lambda i: (i, 0)),
                  pl.BlockSpec((D, D), lambda i: (0, 0))],
        out_specs=pl.BlockSpec((BN, D), lambda i: (i, 0)),
        out_shape=jax.ShapeDtypeStruct((N_NODES, D), jnp.float32),
    )(node_feats, W_up)

    BE = 6400
    NBLK_H = E_H // BE
    efT, eaT = edge_feats.T, edge_attrs.T

    def _mlp_half(half):
        return pl.pallas_call(
            _mlp_body,
            grid=(NBLK_H,),
            in_specs=[pl.BlockSpec((8, BE), lambda i: (0, i + half * NBLK_H)),
                      pl.BlockSpec((1, BE), lambda i: (0, i + half * NBLK_H)),
                      pl.BlockSpec((8, 64), lambda i: (0, 0)),
                      pl.BlockSpec((64, 64), lambda i: (0, 0)),
                      pl.BlockSpec((64, 64), lambda i: (0, 0)),
                      pl.BlockSpec((64, D), lambda i: (0, 0))],
            out_specs=pl.BlockSpec((BE, D), lambda i: (i, 0)),
            out_shape=jax.ShapeDtypeStruct((E_H, D), jnp.float32),
        )(efT, eaT, mlp_W0, mlp_W1, mlp_W2, mlp_W3)

    zeros_blk = jnp.zeros((ROWS_PER, D), jnp.float32)
    ei_flat = edge_index.reshape(2 * E)

    tp2_a = _mlp_half(0)
    partials_a = _sc_scatter_a(ei_flat, h, tp2_a, zeros_blk)
    tp2_b = _mlp_half(1)
    partials_b = _sc_scatter_b(ei_flat, h, tp2_b, zeros_blk)

    W_skip_t = jnp.transpose(W_skip, (1, 0, 2))
    BF = 2000
    out = pl.pallas_call(
        _final_body,
        grid=(N_NODES // BF,),
        in_specs=[pl.BlockSpec((1, BF, D), lambda i: (0, i, 0)),
                  pl.BlockSpec((1, BF, D), lambda i: (1, i, 0)),
                  pl.BlockSpec((1, BF, D), lambda i: (0, i, 0)),
                  pl.BlockSpec((1, BF, D), lambda i: (1, i, 0)),
                  pl.BlockSpec((BF, DA), lambda i: (i, 0)),
                  pl.BlockSpec((D, D), lambda i: (0, 0)),
                  pl.BlockSpec((DA, D, D), lambda i: (0, 0, 0))],
        out_specs=pl.BlockSpec((BF, D), lambda i: (i, 0)),
        out_shape=jax.ShapeDtypeStruct((N_NODES, D), jnp.float32),
    )(partials_a, partials_a, partials_b, partials_b,
      node_attrs, W_lin, W_skip_t)

    return out.reshape(N_NODES, D, 1)


# speedup vs baseline: 1.0896x; 1.0047x over previous
import functools

import jax
import jax.numpy as jnp
from jax import lax
from jax.experimental import pallas as pl
from jax.experimental.pallas import tpu as pltpu
from jax.experimental.pallas import tpu_sc as plsc

N_NODES = 10000
D = 128
DA = 16
E = 320000

NC = 2
NS = 16
NW = NC * NS
E_H = E // 2
E_PER = E_H // NW
CH = 40
NCH = E_PER // CH
NBUF = 4
APAD = 10240
ROWS_PER = APAD // NS

_INV_SQRT_D = 1.0 / (D ** 0.5)
_INV_SQRT8 = 1.0 / (8.0 ** 0.5)
_INV_SQRT64 = 1.0 / 8.0


def _silu(x):
    return (0.5 * x) * (1.0 + jnp.tanh(0.5 * x))


def _h_body(nf, w, o):
    o[...] = jnp.dot(nf[...], w[...], preferred_element_type=jnp.float32) * _INV_SQRT_D


def _mlp_body(xT, eaT, w0, w1, w2, w3, o):
    a = lax.dot_general(xT[...], w0[...], (((0,), (0,)), ((), ())),
                        preferred_element_type=jnp.float32)
    a = _silu(a * _INV_SQRT8)
    a = _silu(jnp.dot(a, w1[...], preferred_element_type=jnp.float32) * _INV_SQRT64)
    a = _silu(jnp.dot(a, w2[...], preferred_element_type=jnp.float32) * _INV_SQRT64)
    a = a * jnp.transpose(eaT[...])
    o[...] = jnp.dot(a, w3[...], preferred_element_type=jnp.float32) * _INV_SQRT64


def _final_body(p0, p1, at, wl, ws, o):
    msg = jnp.dot(p0[0] + p1[0], wl[...],
                  preferred_element_type=jnp.float32)
    msg = msg * (_INV_SQRT_D / 32.0)
    attrs = at[...]
    acc = jnp.zeros(msg.shape, jnp.float32)
    for v in range(DA):
        acc = acc + jnp.dot(msg * attrs[:, v:v + 1], ws[v],
                            preferred_element_type=jnp.float32)
    o[...] = acc * (1.0 / ((D * DA) ** 0.5))


def _sc_scatter_body(half, ei_hbm, h_hbm, tp2_hbm, init_hbm, out_hbm,
                     srb, rows, tp2v, acc, isems, gsems, tsems, ssems):
    cid = lax.axis_index("c")
    sid = lax.axis_index("s")
    wid = sid * NC + cid

    rs = pl.ds(sid * ROWS_PER, ROWS_PER)
    if half == 0:
        pltpu.sync_copy(init_hbm, acc.at[rs])
    else:
        pltpu.sync_copy(init_hbm.at[cid, rs], acc.at[rs])
    plsc.subcore_barrier()

    tbase = wid * E_PER
    ebase = half * E_H + tbase

    def mul(b):
        @plsc.parallel_loop(0, CH, unroll=2)
        def _(r):
            for j in range(D // 16):
                s = pl.ds(j * 16, 16)
                rows[b, r, s] = rows[b, r, s] * tp2v[b, r, s]

    def stage_a(b, c, first=False):
        if not first:
            pltpu.make_async_copy(rows.at[b], acc.at[srb.at[b, 0]],
                                  ssems.at[b]).wait()
        pltpu.async_copy(ei_hbm.at[pl.ds(E + ebase + c * CH, CH)],
                         srb.at[b, 0], isems.at[b])
        pltpu.async_copy(ei_hbm.at[pl.ds(ebase + c * CH, CH)],
                         srb.at[b, 1], isems.at[b])
        pltpu.async_copy(tp2_hbm.at[pl.ds(tbase + c * CH, CH)], tp2v.at[b],
                         tsems.at[b])

    def stage_b(b, c):
        pltpu.make_async_copy(ei_hbm.at[pl.ds(0, CH)], srb.at[b, 0],
                              isems.at[b]).wait()
        pltpu.make_async_copy(ei_hbm.at[pl.ds(0, CH)], srb.at[b, 1],
                              isems.at[b]).wait()
        pltpu.async_copy(h_hbm.at[srb.at[b, 1]], rows.at[b], gsems.at[b])

    def stage_c(b):
        pltpu.make_async_copy(h_hbm.at[srb.at[b, 1]], rows.at[b],
                              gsems.at[b]).wait()
        pltpu.make_async_copy(tp2_hbm.at[pl.ds(0, CH)], tp2v.at[b],
                              tsems.at[b]).wait()
        mul(b)
        pltpu.async_copy(rows.at[b], acc.at[srb.at[b, 0]], ssems.at[b],
                         add=True)

    stage_a(0, 0, first=True)
    stage_a(1, 1, first=True)
    stage_b(0, 0)
    stage_a(2, 2, first=True)
    stage_b(1, 1)
    stage_c(0)
    stage_a(3, 3, first=True)
    stage_b(2, 2)
    stage_c(1)

    def quad(i, _):
        t = 4 * i + 2
        stage_a(0, t + 2)
        stage_b(3, t + 1)
        stage_c(2)
        stage_a(1, t + 3)
        stage_b(0, t + 2)
        stage_c(3)
        stage_a(2, t + 4)
        stage_b(1, t + 3)
        stage_c(0)
        stage_a(3, t + 5)
        stage_b(2, t + 4)
        stage_c(1)
        return 0

    lax.fori_loop(0, 30, quad, 0)

    stage_a(0, 124)
    stage_b(3, 123)
    stage_c(2)
    stage_b(0, 124)
    stage_c(3)
    stage_c(0)
    for b in (1, 2, 3, 0):
        pltpu.make_async_copy(rows.at[b], acc.at[srb.at[b, 0]],
                              ssems.at[b]).wait()

    plsc.subcore_barrier()
    pltpu.sync_copy(acc.at[pl.ds(sid * ROWS_PER, ROWS_PER)],
                    out_hbm.at[cid, pl.ds(sid * ROWS_PER, ROWS_PER)])


def _make_sc_scatter(half):
    return functools.partial(
        pl.kernel,
        out_type=jax.ShapeDtypeStruct((NC, APAD, D), jnp.float32),
        mesh=plsc.VectorSubcoreMesh(core_axis_name="c", subcore_axis_name="s",
                                    num_cores=NC, num_subcores=NS),
        scratch_types=[
            pltpu.VMEM((NBUF, 2, CH), jnp.int32),
            pltpu.VMEM((NBUF, CH, D), jnp.float32),
            pltpu.VMEM((NBUF, CH, D), jnp.float32),
            pltpu.VMEM_SHARED((APAD, D), jnp.float32),
            pltpu.SemaphoreType.DMA((NBUF,)),
            pltpu.SemaphoreType.DMA((NBUF,)),
            pltpu.SemaphoreType.DMA((NBUF,)),
            pltpu.SemaphoreType.DMA((NBUF,)),
        ],
    )(functools.partial(_sc_scatter_body, half))


_sc_scatter_a = _make_sc_scatter(0)
_sc_scatter_b = _make_sc_scatter(1)


def kernel(node_attrs, node_feats, edge_attrs, edge_feats, edge_index,
           W_up, mlp_W0, mlp_W1, mlp_W2, mlp_W3, W_lin, W_skip):
    BN = 2000
    h = pl.pallas_call(
        _h_body,
        grid=(N_NODES // BN,),
        in_specs=[pl.BlockSpec((BN, D), lambda i: (i, 0)),
                  pl.BlockSpec((D, D), lambda i: (0, 0))],
        out_specs=pl.BlockSpec((BN, D), lambda i: (i, 0)),
        out_shape=jax.ShapeDtypeStruct((N_NODES, D), jnp.float32),
    )(node_feats, W_up)

    BE = 6400
    NBLK_H = E_H // BE
    efT, eaT = edge_feats.T, edge_attrs.T

    def _mlp_half(half):
        return pl.pallas_call(
            _mlp_body,
            grid=(NBLK_H,),
            in_specs=[pl.BlockSpec((8, BE), lambda i: (0, i + half * NBLK_H)),
                      pl.BlockSpec((1, BE), lambda i: (0, i + half * NBLK_H)),
                      pl.BlockSpec((8, 64), lambda i: (0, 0)),
                      pl.BlockSpec((64, 64), lambda i: (0, 0)),
                      pl.BlockSpec((64, 64), lambda i: (0, 0)),
                      pl.BlockSpec((64, D), lambda i: (0, 0))],
            out_specs=pl.BlockSpec((BE, D), lambda i: (i, 0)),
            out_shape=jax.ShapeDtypeStruct((E_H, D), jnp.float32),
        )(efT, eaT, mlp_W0, mlp_W1, mlp_W2, mlp_W3)

    zeros_blk = jnp.zeros((ROWS_PER, D), jnp.float32)
    ei_flat = edge_index.reshape(2 * E)

    tp2_a = _mlp_half(0)
    partials_a = _sc_scatter_a(ei_flat, h, tp2_a, zeros_blk)
    tp2_b = _mlp_half(1)
    partials = _sc_scatter_b(ei_flat, h, tp2_b, partials_a)

    W_skip_t = jnp.transpose(W_skip, (1, 0, 2))
    BF = 2000
    out = pl.pallas_call(
        _final_body,
        grid=(N_NODES // BF,),
        in_specs=[pl.BlockSpec((1, BF, D), lambda i: (0, i, 0)),
                  pl.BlockSpec((1, BF, D), lambda i: (1, i, 0)),
                  pl.BlockSpec((BF, DA), lambda i: (i, 0)),
                  pl.BlockSpec((D, D), lambda i: (0, 0)),
                  pl.BlockSpec((DA, D, D), lambda i: (0, 0, 0))],
        out_specs=pl.BlockSpec((BF, D), lambda i: (i, 0)),
        out_shape=jax.ShapeDtypeStruct((N_NODES, D), jnp.float32),
    )(partials, partials, node_attrs, W_lin, W_skip_t)

    return out.reshape(N_NODES, D, 1)
